# baseline (speedup 1.0000x reference)
import functools

import jax
import jax.numpy as jnp
from jax import lax
from jax.experimental import pallas as pl
from jax.experimental.pallas import tpu as pltpu
from jax.experimental.pallas import tpu_sc as plsc

N = 10000
NPAD = 10240
E = 320000
D = 128
NC, NS = 2, 16
NW = NC * NS
RPT = NPAD // NS

EW = E // NW
CH_D = 125
NCH_D = EW // CH_D
CH = 100
NCH = EW // CH

_mesh = plsc.VectorSubcoreMesh(core_axis_name="c", subcore_axis_name="s")


@functools.partial(
    pl.kernel,
    out_type=jax.ShapeDtypeStruct((NC, 2, NPAD), jnp.float32),
    mesh=_mesh,
    scratch_types=[
        pltpu.VMEM((NCH_D, CH_D), jnp.int32),
        pltpu.VMEM((NCH_D, CH_D), jnp.int32),
        pltpu.VMEM((128,), jnp.float32),
        pltpu.VMEM((RPT,), jnp.float32),
        pltpu.VMEM_SHARED((NPAD,), jnp.float32),
        pltpu.VMEM_SHARED((NPAD,), jnp.float32),
        pltpu.SemaphoreType.DMA,
        pltpu.SemaphoreType.DMA,
    ],
)
def _deg_kernel(ei_hbm, out_hbm, src_v, dst_v, ones_v, zero_v, ho_sh, hi_sh,
                dsem0, dsem1):
    c = lax.axis_index("c")
    s = lax.axis_index("s")
    wid = s * NC + c
    pltpu.sync_copy(ei_hbm.at[0, wid], src_v)
    pltpu.sync_copy(ei_hbm.at[1, wid], dst_v)

    def fill1(i, carry):
        ones_v[pl.ds(i * 16, 16)] = jnp.full((16,), 1.0, jnp.float32)
        return carry

    lax.fori_loop(0, 128 // 16, fill1, 0)

    def fill0(i, carry):
        zero_v[pl.ds(i * 16, 16)] = jnp.zeros((16,), jnp.float32)
        return carry

    lax.fori_loop(0, RPT // 16, fill0, 0)

    pltpu.sync_copy(zero_v, ho_sh.at[pl.ds(s * RPT, RPT)])
    pltpu.sync_copy(zero_v, hi_sh.at[pl.ds(s * RPT, RPT)])
    plsc.subcore_barrier()

    ones_ch = ones_v.at[pl.ds(0, CH_D)]

    def body(k, carry):
        pltpu.async_copy(ones_ch, ho_sh.at[src_v.at[k]], dsem0, add=True)
        pltpu.async_copy(ones_ch, hi_sh.at[dst_v.at[k]], dsem1, add=True)
        return carry

    lax.fori_loop(0, NCH_D, body, 0)

    def drain(k, carry):
        pltpu.make_async_copy(ones_ch, ho_sh.at[src_v.at[0]], dsem0).wait()
        pltpu.make_async_copy(ones_ch, hi_sh.at[dst_v.at[0]], dsem1).wait()
        return carry

    lax.fori_loop(0, NCH_D, drain, 0)
    plsc.subcore_barrier()

    pltpu.sync_copy(ho_sh.at[pl.ds(s * RPT, RPT)],
                    out_hbm.at[c, 0, pl.ds(s * RPT, RPT)])
    pltpu.sync_copy(hi_sh.at[pl.ds(s * RPT, RPT)],
                    out_hbm.at[c, 1, pl.ds(s * RPT, RPT)])


def _norm_body(deg_ref, x_ref, feat_ref, ninb_ref):
    p = deg_ref[...]
    do = p[0, 0, :N] + p[1, 0, :N]
    di = p[0, 1, :N] + p[1, 1, :N]
    no = lax.rsqrt(jnp.maximum(do, 1.0))
    ni = lax.rsqrt(jnp.maximum(di, 1.0))
    feat_ref[...] = x_ref[...] * no[:, None]
    ninb_ref[...] = jnp.broadcast_to(ni[:, None], (N, D))


_norm_call = pl.pallas_call(
    _norm_body,
    out_shape=[
        jax.ShapeDtypeStruct((N, D), jnp.float32),
        jax.ShapeDtypeStruct((N, D), jnp.float32),
    ],
)


@functools.partial(
    pl.kernel,
    out_type=jax.ShapeDtypeStruct((NC, NPAD, D), jnp.float32),
    mesh=_mesh,
    scratch_types=[
        pltpu.VMEM((NCH, CH), jnp.int32),
        pltpu.VMEM((NCH, CH), jnp.int32),
        pltpu.VMEM((2, CH, D), jnp.float32),
        pltpu.VMEM_SHARED((NPAD, D), jnp.float32),
        [pltpu.SemaphoreType.DMA] * 2,
        [pltpu.SemaphoreType.DMA] * 2,
    ],
    compiler_params=pltpu.CompilerParams(use_tc_tiling_on_sc=False),
)
def _agg_kernel(feat_hbm, ei_hbm, zc_hbm, out_hbm, src_v, dst_v, rows_v,
                agg_sh, gsems, ssems):
    c = lax.axis_index("c")
    s = lax.axis_index("s")
    wid = s * NC + c
    pltpu.sync_copy(ei_hbm.at[0, wid], src_v)
    pltpu.sync_copy(ei_hbm.at[1, wid], dst_v)

    def zc(j, carry):
        pltpu.sync_copy(zc_hbm, agg_sh.at[pl.ds(s * RPT + j * 128, 128)])
        return carry

    lax.fori_loop(0, RPT // 128, zc, 0)
    plsc.subcore_barrier()

    def start_gather(k, b):
        pltpu.async_copy(feat_hbm.at[src_v.at[k]], rows_v.at[b], gsems[b])

    def wait_gather(k, b):
        pltpu.make_async_copy(feat_hbm.at[src_v.at[k]], rows_v.at[b],
                              gsems[b]).wait()

    def start_scatter(k, b):
        pltpu.async_copy(rows_v.at[b], agg_sh.at[dst_v.at[k]], ssems[b],
                         add=True)

    def wait_scatter(b):
        pltpu.make_async_copy(rows_v.at[b], agg_sh.at[dst_v.at[0]],
                              ssems[b]).wait()

    start_gather(0, 0)

    def step(k2, carry):
        for b in range(2):
            k = k2 * 2 + b
            nb = (b + 1) % 2
            wait_gather(k, b)
            start_scatter(k, b)

            @pl.when(k + 1 < NCH)
            def _():
                @pl.when(k >= 1)
                def _():
                    wait_scatter(nb)

                start_gather(k + 1, nb)
        return carry

    lax.fori_loop(0, NCH // 2, step, 0)
    wait_scatter(0)
    wait_scatter(1)
    plsc.subcore_barrier()

    pltpu.sync_copy(agg_sh.at[pl.ds(s * RPT, RPT)],
                    out_hbm.at[c, pl.ds(s * RPT, RPT)])


def _out_body(pa_ref, ninb_ref, w_ref, b_ref, o_ref):
    agg = pa_ref[0] + pa_ref[1]
    rst = agg * ninb_ref[...]
    o_ref[...] = (
        jnp.dot(rst, w_ref[...], preferred_element_type=jnp.float32)
        + b_ref[...]
    )


_BLK = 1000
_out_call = pl.pallas_call(
    _out_body,
    grid=(N // _BLK,),
    in_specs=[
        pl.BlockSpec((NC, _BLK, D), lambda i: (0, i, 0)),
        pl.BlockSpec((_BLK, D), lambda i: (i, 0)),
        pl.BlockSpec((D, D), lambda i: (0, 0)),
        pl.BlockSpec((1, D), lambda i: (0, 0)),
    ],
    out_specs=pl.BlockSpec((_BLK, D), lambda i: (i, 0)),
    out_shape=jax.ShapeDtypeStruct((N, D), jnp.float32),
)


def kernel(x, edge_index, W, b):
    ei32 = edge_index.astype(jnp.int32)
    ei_d = ei32.reshape(2, NW, NCH_D, CH_D)
    ei_a = ei32.reshape(2, NW, NCH, CH)
    zc = jnp.zeros((128, D), jnp.float32)
    deg = _deg_kernel(ei_d)
    feat, ninb = _norm_call(deg, x)
    pagg = _agg_kernel(feat, ei_a, zc)
    return _out_call(pagg, ninb, W, b.reshape(1, D))
